# baseline (speedup 1.0000x reference)
import jax
import jax.numpy as jnp
from jax import lax
from jax.experimental import pallas as pl
from jax.experimental.pallas import tpu as pltpu
from jax.experimental.pallas import tpu_sc as plsc

_E_IND, _E_SH = 8, 3
_EL = 16
_BT = 512
_NB = 72
_NW = 32
_CH = 128
_CT = 64


def _gate_shared_kernel(x_ref, obj_ref, gW_ref, gb_ref, sW1_ref, sb1_ref,
                        sW2_ref, sb2_ref,
                        sh_ref, i1_ref, i2_ref, w1_ref, w2_ref,
                        r1_ref, r2_ref, cnt_ref):
    x = x_ref[...]
    T = x.shape[0]

    logits = jnp.dot(x, gW_ref[...], preferred_element_type=jnp.float32)
    logits = logits + gb_ref[0]
    iota_e = lax.broadcasted_iota(jnp.int32, logits.shape, 1)
    m1 = jnp.max(logits, axis=1, keepdims=True)
    i1 = jnp.min(jnp.where(logits == m1, iota_e, _E_IND), axis=1,
                 keepdims=True)
    l2 = jnp.where(iota_e == i1, -jnp.inf, logits)
    m2 = jnp.max(l2, axis=1, keepdims=True)
    i2 = jnp.min(jnp.where(l2 == m2, iota_e, _E_IND), axis=1, keepdims=True)
    w1 = 0.5 / (1.0 + jnp.exp(m2 - m1))
    w2 = 0.5 - w1

    lane = lax.broadcasted_iota(jnp.int32, (T, _EL), 1)
    oh1 = (lane == i1).astype(jnp.float32)
    oh2 = (lane == i2).astype(jnp.float32)
    row = lax.broadcasted_iota(jnp.int32, (T, T), 0)
    col = lax.broadcasted_iota(jnp.int32, (T, T), 1)
    tri = (row > col).astype(jnp.float32)
    ex1 = jnp.dot(tri, oh1, preferred_element_type=jnp.float32)
    ex2 = jnp.dot(tri, oh2, preferred_element_type=jnp.float32)
    c1 = jnp.sum(oh1, axis=0, keepdims=True)
    c2 = jnp.sum(oh2, axis=0, keepdims=True)
    r1 = jnp.sum(jnp.where(lane == i1, ex1, 0.0), axis=1, keepdims=True)
    r2 = jnp.sum(jnp.where(lane == i2, ex2 + c1, 0.0), axis=1, keepdims=True)

    i1_ref[...] = i1
    i2_ref[...] = i2
    w1_ref[...] = w1
    w2_ref[...] = w2
    r1_ref[...] = r1.astype(jnp.int32)
    r2_ref[...] = r2.astype(jnp.int32)
    cnt_ref[...] = (c1 + c2).astype(jnp.int32)[None]

    obj = obj_ref[0]
    acc = jnp.zeros((T, sh_ref.shape[1]), dtype=jnp.float32)
    for s in range(_E_SH):
        h = jnp.maximum(
            jnp.dot(x, sW1_ref[s], preferred_element_type=jnp.float32)
            + sb1_ref[s], 0.0)
        o = jnp.dot(h, sW2_ref[s], preferred_element_type=jnp.float32) \
            + sb2_ref[s]
        acc = acc + o * obj[:, s:s + 1]
    sh_ref[...] = 0.5 * acc


def _take16(vec, idx):
    dnums = lax.GatherDimensionNumbers(
        offset_dims=(), collapsed_slice_dims=(0,), start_index_map=(0,))
    return lax.gather(vec, idx[:, None], dnums, (1,),
                      mode=lax.GatherScatterMode.PROMISE_IN_BOUNDS)


def _sc_dispatch_body(x_hbm, i1_hbm, i2_hbm, r1_hbm, r2_hbm, w1_hbm, w2_hbm,
                      tb_hbm,
                      xs_hbm, ws_hbm, pos1_hbm, pos2_hbm,
                      i1_v, i2_v, r1_v, r2_v, w1_v, w2_v, tb_v,
                      d1_v, d2_v, id_v, rows_v, sem):
    nc = 2
    wid = lax.axis_index("s") * nc + lax.axis_index("c")
    tpw = i1_v.shape[0]
    base = wid * tpw

    pltpu.sync_copy(tb_hbm.at[wid], tb_v)
    pltpu.sync_copy(i1_hbm.at[pl.ds(base, tpw)], i1_v)
    pltpu.sync_copy(i2_hbm.at[pl.ds(base, tpw)], i2_v)
    pltpu.sync_copy(r1_hbm.at[pl.ds(base, tpw)], r1_v)
    pltpu.sync_copy(r2_hbm.at[pl.ds(base, tpw)], r2_v)
    pltpu.sync_copy(w1_hbm.at[wid], w1_v)
    pltpu.sync_copy(w2_hbm.at[wid], w2_v)

    lane = lax.iota(jnp.int32, 16)
    tb = tb_v[...]

    nv = tpw // 16
    for k in range(nv):
        sl = pl.ds(k * 16, 16)
        d1 = r1_v[sl] + _take16(tb, i1_v[sl])
        d2 = r2_v[sl] + _take16(tb, i2_v[sl])
        csl = pl.ds((k % 8) * 16, 16)
        d1_v[k // 8, csl] = d1
        d2_v[k // 8, csl] = d2
        id_v[k // 8, csl] = base + k * 16 + lane

    pltpu.sync_copy(d1_v, pos1_hbm.at[wid])
    pltpu.sync_copy(d2_v, pos2_hbm.at[wid])

    for c in range(4):
        pltpu.async_copy(w1_v.at[c], ws_hbm.at[d1_v.at[c]], sem).wait()
        pltpu.async_copy(w2_v.at[c], ws_hbm.at[d2_v.at[c]], sem).wait()

    for c in range(8):
        pltpu.async_copy(x_hbm.at[id_v.at[c % 4]], rows_v, sem).wait()
        dst = d1_v.at[c % 4] if c < 4 else d2_v.at[c - 4]
        pltpu.async_copy(rows_v, xs_hbm.at[dst], sem).wait()


def _grouped_mm_kernel(be_ref, xs_ref, w_ref, iW1_ref, ib1_ref, iW2_ref,
                       ib2_ref, o_ref):
    x = xs_ref[...]
    h = jnp.maximum(
        jnp.dot(x, iW1_ref[0], preferred_element_type=jnp.float32)
        + ib1_ref[0, 0], 0.0)
    o = jnp.dot(h, iW2_ref[0], preferred_element_type=jnp.float32) \
        + ib2_ref[0, 0]
    o_ref[...] = o * w_ref[...].reshape(-1, 1)


def _sc_combine_body(sh_hbm, os_hbm, pos1_hbm, pos2_hbm,
                     out_hbm,
                     p1_v, p2_v, acc_v, g_v, sem):
    nc = 2
    wid = lax.axis_index("s") * nc + lax.axis_index("c")
    tpw = 512
    base = wid * tpw

    pltpu.sync_copy(pos1_hbm.at[wid], p1_v)
    pltpu.sync_copy(pos2_hbm.at[wid], p2_v)

    nvec = acc_v.shape[1] // 16

    def _addrow(r, _):
        for v in range(nvec):
            sl = pl.ds(v * 16, 16)
            acc_v[r, sl] = acc_v[r, sl] + g_v[r, sl]
        return 0

    for c in range(tpw // _CT):
        tok0 = base + c * _CT
        pltpu.sync_copy(sh_hbm.at[pl.ds(tok0, _CT)], acc_v)
        idx1 = p1_v.at[c // 2, pl.ds((c % 2) * _CT, _CT)]
        pltpu.async_copy(os_hbm.at[idx1], g_v, sem).wait()
        lax.fori_loop(0, _CT, _addrow, 0)
        idx2 = p2_v.at[c // 2, pl.ds((c % 2) * _CT, _CT)]
        pltpu.async_copy(os_hbm.at[idx2], g_v, sem).wait()
        lax.fori_loop(0, _CT, _addrow, 0)
        pltpu.sync_copy(acc_v, out_hbm.at[pl.ds(tok0, _CT)])


def kernel(feature_vectors, object_types, gW, gb, sW1, sb1, sW2, sb2,
           iW1, ib1, iW2, ib2):
    B, NN, HL, D = feature_vectors.shape
    N = B * NN * HL
    H = sW1.shape[-1]
    O = sW2.shape[-1]
    x = feature_vectors.reshape(N, D)
    T = _BT
    nblk = N // T
    obj = object_types.reshape(nblk, T, 3)
    gb2 = gb.reshape(1, -1)
    padn = _NB * _BT

    full = lambda a: pl.BlockSpec(a.shape, lambda i: (0,) * a.ndim)
    sh, i1, i2, w1, w2, r1, r2, cnt = pl.pallas_call(
        _gate_shared_kernel,
        grid=(nblk,),
        in_specs=[
            pl.BlockSpec((T, D), lambda i: (i, 0)),
            pl.BlockSpec((1, T, 3), lambda i: (i, 0, 0)),
            full(gW), full(gb2), full(sW1), full(sb1), full(sW2), full(sb2),
        ],
        out_specs=[
            pl.BlockSpec((T, O), lambda i: (i, 0)),
            pl.BlockSpec((T, 1), lambda i: (i, 0)),
            pl.BlockSpec((T, 1), lambda i: (i, 0)),
            pl.BlockSpec((T, 1), lambda i: (i, 0)),
            pl.BlockSpec((T, 1), lambda i: (i, 0)),
            pl.BlockSpec((T, 1), lambda i: (i, 0)),
            pl.BlockSpec((T, 1), lambda i: (i, 0)),
            pl.BlockSpec((1, 1, _EL), lambda i: (i, 0, 0)),
        ],
        out_shape=[
            jax.ShapeDtypeStruct((N, O), jnp.float32),
            jax.ShapeDtypeStruct((N, 1), jnp.int32),
            jax.ShapeDtypeStruct((N, 1), jnp.int32),
            jax.ShapeDtypeStruct((N, 1), jnp.float32),
            jax.ShapeDtypeStruct((N, 1), jnp.float32),
            jax.ShapeDtypeStruct((N, 1), jnp.int32),
            jax.ShapeDtypeStruct((N, 1), jnp.int32),
            jax.ShapeDtypeStruct((nblk, 1, _EL), jnp.int32),
        ],
        compiler_params=pltpu.CompilerParams(
            dimension_semantics=("arbitrary",)),
    )(x, obj, gW, gb2, sW1, sb1, sW2, sb2)

    c = cnt[:, 0, :]
    tot = jnp.sum(c, axis=0)
    padded = ((tot + _BT - 1) // _BT) * _BT
    pad_cum = jnp.cumsum(padded)
    pad_off = pad_cum - padded
    tile_base = (pad_off[None, :] + jnp.cumsum(c, axis=0) - c).astype(
        jnp.int32)
    block_expert = jnp.minimum(
        jnp.searchsorted(pad_cum, jnp.arange(_NB, dtype=jnp.int32) * _BT,
                         side='right'),
        _E_IND - 1).astype(jnp.int32)

    tpw = N // _NW
    mesh = plsc.VectorSubcoreMesh(core_axis_name="c", subcore_axis_name="s")
    xs, ws, pos1, pos2 = pl.kernel(
        _sc_dispatch_body,
        out_type=[
            jax.ShapeDtypeStruct((padn, D), jnp.float32),
            jax.ShapeDtypeStruct((padn,), jnp.float32),
            jax.ShapeDtypeStruct((_NW, 4, _CH), jnp.int32),
            jax.ShapeDtypeStruct((_NW, 4, _CH), jnp.int32),
        ],
        mesh=mesh,
        scratch_types=[
            pltpu.VMEM((tpw,), jnp.int32),
            pltpu.VMEM((tpw,), jnp.int32),
            pltpu.VMEM((tpw,), jnp.int32),
            pltpu.VMEM((tpw,), jnp.int32),
            pltpu.VMEM((4, _CH), jnp.float32),
            pltpu.VMEM((4, _CH), jnp.float32),
            pltpu.VMEM((_EL,), jnp.int32),
            pltpu.VMEM((4, _CH), jnp.int32),
            pltpu.VMEM((4, _CH), jnp.int32),
            pltpu.VMEM((4, _CH), jnp.int32),
            pltpu.VMEM((_CH, D), jnp.float32),
            pltpu.SemaphoreType.DMA,
        ],
    )(x, i1.reshape(N), i2.reshape(N), r1.reshape(N), r2.reshape(N),
      w1.reshape(_NW, 4, _CH), w2.reshape(_NW, 4, _CH), tile_base)

    osort = pl.pallas_call(
        _grouped_mm_kernel,
        grid_spec=pltpu.PrefetchScalarGridSpec(
            num_scalar_prefetch=1,
            grid=(_NB,),
            in_specs=[
                pl.BlockSpec((_BT, D), lambda i, be: (i, 0)),
                pl.BlockSpec((_BT,), lambda i, be: (i,)),
                pl.BlockSpec((1, D, H), lambda i, be: (be[i], 0, 0)),
                pl.BlockSpec((1, 1, H), lambda i, be: (be[i], 0, 0)),
                pl.BlockSpec((1, H, O), lambda i, be: (be[i], 0, 0)),
                pl.BlockSpec((1, 1, O), lambda i, be: (be[i], 0, 0)),
            ],
            out_specs=pl.BlockSpec((_BT, O), lambda i, be: (i, 0)),
        ),
        out_shape=jax.ShapeDtypeStruct((padn, O), jnp.float32),
        compiler_params=pltpu.CompilerParams(
            dimension_semantics=("arbitrary",)),
    )(block_expert, xs, ws, iW1, ib1.reshape(_E_IND, 1, H),
      iW2, ib2.reshape(_E_IND, 1, O))

    out = pl.kernel(
        _sc_combine_body,
        out_type=jax.ShapeDtypeStruct((N, O), jnp.float32),
        mesh=mesh,
        scratch_types=[
            pltpu.VMEM((4, _CH), jnp.int32),
            pltpu.VMEM((4, _CH), jnp.int32),
            pltpu.VMEM((_CT, O), jnp.float32),
            pltpu.VMEM((_CT, O), jnp.float32),
            pltpu.SemaphoreType.DMA,
        ],
    )(sh, osort, pos1, pos2)

    return out.reshape(B, NN, HL, O)


# speedup vs baseline: 3.0011x; 3.0011x over previous
import jax
import jax.numpy as jnp
from jax.experimental import pallas as pl
from jax.experimental.pallas import tpu as pltpu

_E_IND, _E_SH = 8, 3


def _moe_block_kernel(x_ref, obj_ref, gW_ref, gb_ref, sW1_ref, sb1_ref,
                      sW2_ref, sb2_ref, iW1_ref, ib1_ref, iW2_ref, ib2_ref,
                      out_ref):
    x = x_ref[...]
    T = x.shape[0]

    logits = jnp.dot(x, gW_ref[...], preferred_element_type=jnp.float32)
    logits = logits + gb_ref[0]
    iota = jax.lax.broadcasted_iota(jnp.int32, logits.shape, 1)
    m1 = jnp.max(logits, axis=1, keepdims=True)
    i1 = jnp.min(jnp.where(logits == m1, iota, _E_IND), axis=1, keepdims=True)
    l2 = jnp.where(iota == i1, -jnp.inf, logits)
    m2 = jnp.max(l2, axis=1, keepdims=True)
    i2 = jnp.min(jnp.where(l2 == m2, iota, _E_IND), axis=1, keepdims=True)
    w1 = 1.0 / (1.0 + jnp.exp(m2 - m1))
    w2 = 1.0 - w1

    obj = obj_ref[0]

    acc = jnp.zeros((T, out_ref.shape[1]), dtype=jnp.float32)
    for s in range(_E_SH):
        h = jnp.maximum(
            jnp.dot(x, sW1_ref[s], preferred_element_type=jnp.float32)
            + sb1_ref[s], 0.0)
        o = jnp.dot(h, sW2_ref[s], preferred_element_type=jnp.float32) \
            + sb2_ref[s]
        acc = acc + o * obj[:, s:s + 1]

    for e in range(_E_IND):
        ce = w1 * (i1 == e) + w2 * (i2 == e)
        h = jnp.maximum(
            jnp.dot(x, iW1_ref[e], preferred_element_type=jnp.float32)
            + ib1_ref[e], 0.0)
        o = jnp.dot(h, iW2_ref[e], preferred_element_type=jnp.float32) \
            + ib2_ref[e]
        acc = acc + o * ce

    out_ref[...] = 0.5 * acc


def kernel(feature_vectors, object_types, gW, gb, sW1, sb1, sW2, sb2,
           iW1, ib1, iW2, ib2):
    B, NN, HL, D = feature_vectors.shape
    N = B * NN * HL
    O = sW2.shape[-1]
    x = feature_vectors.reshape(N, D)
    T = 512 if N % 512 == 0 else (256 if N % 256 == 0 else N)
    nblk = N // T
    obj = object_types.reshape(nblk, T, 3)
    gb2 = gb.reshape(1, -1)

    full = lambda a: pl.BlockSpec(a.shape, lambda i: (0,) * a.ndim)
    out = pl.pallas_call(
        _moe_block_kernel,
        grid=(nblk,),
        in_specs=[
            pl.BlockSpec((T, D), lambda i: (i, 0)),
            pl.BlockSpec((1, T, 3), lambda i: (i, 0, 0)),
            full(gW), full(gb2), full(sW1), full(sb1), full(sW2), full(sb2),
            full(iW1), full(ib1), full(iW2), full(ib2),
        ],
        out_specs=pl.BlockSpec((T, O), lambda i: (i, 0)),
        out_shape=jax.ShapeDtypeStruct((N, O), jnp.float32),
        compiler_params=pltpu.CompilerParams(
            dimension_semantics=("parallel",)),
    )(x, obj, gW, gb2, sW1, sb1, sW2, sb2, iW1, ib1, iW2, ib2)
    return out.reshape(B, NN, HL, O)
